# Initial kernel scaffold; baseline (speedup 1.0000x reference)
import jax
import jax.numpy as jnp
from jax.experimental import pallas as pl


def kernel(x, edge_index, batch, params):
    raise NotImplementedError("write your pallas kernel here")


# speedup vs baseline: 4.7113x; 4.7113x over previous
import functools

import jax
import jax.numpy as jnp
from jax import lax
from jax.experimental import pallas as pl
from jax.experimental.pallas import tpu as pltpu
from jax.experimental.pallas import tpu_sc as plsc

N = 10000
D = 128
E = 320000
NC = 2
NS = 16
NW = NC * NS
EPW = E // NW
CHUNK = 80
NCH = EPW // CHUNK
NPAD = 10240
RPT = NPAD // NS


@functools.partial(
    pl.kernel,
    out_type=jax.ShapeDtypeStruct((NC, NPAD, D), jnp.float32),
    mesh=plsc.VectorSubcoreMesh(core_axis_name="c", subcore_axis_name="s"),
    scratch_types=[
        pltpu.VMEM((NCH, CHUNK), jnp.int32),
        pltpu.VMEM((NCH, CHUNK), jnp.int32),
        pltpu.VMEM((CHUNK, D), jnp.float32),
        pltpu.VMEM_SHARED((NPAD, D), jnp.float32),
        pltpu.SemaphoreType.DMA,
    ],
)
def _sc_segment_sum(h_hbm, src_hbm, dst_hbm, zero_hbm, out_hbm,
                    srcv, dstv, buf, acc, sem):
    c = lax.axis_index("c")
    s = lax.axis_index("s")
    wid = s * NC + c
    pltpu.sync_copy(src_hbm.at[wid], srcv)
    pltpu.sync_copy(dst_hbm.at[wid], dstv)
    pltpu.sync_copy(zero_hbm, acc.at[pl.ds(s * RPT, RPT)])
    plsc.subcore_barrier()

    def step(i, carry):
        pltpu.async_copy(h_hbm.at[srcv.at[i]], buf, sem).wait()
        pltpu.sync_copy(buf, acc.at[dstv.at[i]], add=True)
        return carry

    lax.fori_loop(0, NCH, step, 0)
    plsc.subcore_barrier()
    pltpu.sync_copy(acc.at[pl.ds(s * RPT, RPT)],
                    out_hbm.at[c].at[pl.ds(s * RPT, RPT)])


def _tc_layer_body(h_ref, p0_ref, p1_ref, w1_ref, b1_ref, w2_ref, b2_ref,
                   g_ref, be_ref, o_ref):
    m = h_ref[...] + p0_ref[0:N] + p1_ref[0:N]
    m = jnp.maximum(
        jnp.dot(m, w1_ref[...], preferred_element_type=jnp.float32)
        + b1_ref[...], 0.0)
    m = jnp.maximum(
        jnp.dot(m, w2_ref[...], preferred_element_type=jnp.float32)
        + b2_ref[...], 0.0)
    mean = jnp.mean(m, axis=0, keepdims=True)
    cen = m - mean
    var = jnp.mean(cen * cen, axis=0, keepdims=True)
    o_ref[...] = cen * lax.rsqrt(var + 1e-5) * g_ref[...] + be_ref[...]


def _tc_final_body(h_ref, p0_ref, p1_ref, w1_ref, b1_ref, w2_ref, b2_ref,
                   g_ref, be_ref, fw1_ref, fb1_ref, fw2_ref, fb2_ref, o_ref):
    m = h_ref[...] + p0_ref[0:N] + p1_ref[0:N]
    m = jnp.maximum(
        jnp.dot(m, w1_ref[...], preferred_element_type=jnp.float32)
        + b1_ref[...], 0.0)
    m = jnp.maximum(
        jnp.dot(m, w2_ref[...], preferred_element_type=jnp.float32)
        + b2_ref[...], 0.0)
    mean = jnp.mean(m, axis=0, keepdims=True)
    cen = m - mean
    var = jnp.mean(cen * cen, axis=0, keepdims=True)
    hbn = cen * lax.rsqrt(var + 1e-5) * g_ref[...] + be_ref[...]
    t = jnp.maximum(
        jnp.dot(hbn, fw1_ref[...], preferred_element_type=jnp.float32)
        + fb1_ref[...], 0.0)
    o_ref[...] = (jnp.dot(t, fw2_ref[...], preferred_element_type=jnp.float32)
                  + fb2_ref[...])


_tc_layer = pl.pallas_call(
    _tc_layer_body, out_shape=jax.ShapeDtypeStruct((N, D), jnp.float32))

_tc_final = pl.pallas_call(
    _tc_final_body, out_shape=jax.ShapeDtypeStruct((N, 128), jnp.float32))


def kernel(x, edge_index, batch, params):
    layers, fcW1, fcb1, fcW2, fcb2 = params
    dst_s, src_s = lax.sort_key_val(edge_index[1], edge_index[0],
                                    is_stable=True)
    src3 = src_s.reshape(NW, NCH, CHUNK)
    dst3 = dst_s.reshape(NW, NCH, CHUNK)
    zero = jnp.zeros((RPT, D), jnp.float32)
    nclass = fcW2.shape[1]
    fw2p = jnp.zeros((D, 128), jnp.float32).at[:, :nclass].set(fcW2)
    fb2p = jnp.zeros((1, 128), jnp.float32).at[:, :nclass].set(fcb2)

    h = x
    for li, (W1, b1, W2, b2, g, be) in enumerate(layers):
        parts = _sc_segment_sum(h, src3, dst3, zero)
        common = (h, parts[0], parts[1], W1, b1.reshape(1, D), W2,
                  b2.reshape(1, D), g.reshape(1, D), be.reshape(1, D))
        if li < len(layers) - 1:
            h = _tc_layer(*common)
        else:
            out = _tc_final(*common, fcW1, fcb1.reshape(1, D), fw2p, fb2p)
    return out[:, :nclass]
